# baseline (speedup 1.0000x reference)
import functools

import jax
import jax.numpy as jnp
from jax import lax
from jax.experimental import pallas as pl
from jax.experimental.pallas import tpu as pltpu
from jax.experimental.pallas import tpu_sc as plsc

N = 10000
E = 320000
D = 128
H = 128
G = 64

NC = 2
NS = 16
NW = NC * NS
EB = 128
NBPT = 80
NB2 = NBPT * NW
EPAD = NB2 * EB - E
NPAD = 10112
RPT = NPAD // NS
_CHUNKS = [(k * EB, EB) for k in range(RPT // EB)] + \
    ([(RPT - RPT % EB, RPT % EB)] if RPT % EB else [])
DUMP = NPAD - 1

_mesh = plsc.VectorSubcoreMesh(core_axis_name="c", subcore_axis_name="s")


def _wid():
    return lax.axis_index("s") * NC + lax.axis_index("c")


@functools.partial(
    pl.kernel,
    mesh=_mesh,
    out_type=jax.ShapeDtypeStruct((NC, NPAD, H), jnp.float32),
    scratch_types=[
        pltpu.VMEM((EB, H), jnp.float32),
        pltpu.VMEM((NBPT, EB), jnp.int32),
        pltpu.VMEM_SHARED((NPAD, H), jnp.float32),
    ],
)
def _deg_kernel(dst_hbm, ones_hbm, zeros_hbm, cnt_hbm, ones_v, didx_all,
                acc):
    c = lax.axis_index("c")
    s = lax.axis_index("s")
    wid = _wid()
    base = s * RPT
    for off, ln in _CHUNKS:
        pltpu.sync_copy(zeros_hbm.at[pl.ds(0, ln)],
                        acc.at[pl.ds(base + off, ln)])
    pltpu.sync_copy(ones_hbm, ones_v)
    pltpu.sync_copy(dst_hbm.at[pl.ds(wid * NBPT, NBPT)], didx_all)
    plsc.subcore_barrier()

    def body(j, carry):
        pltpu.sync_copy(ones_v, acc.at[didx_all.at[j]], add=True)
        return carry

    lax.fori_loop(0, NBPT, body, 0)
    plsc.subcore_barrier()
    for off, ln in _CHUNKS:
        sl = pl.ds(base + off, ln)
        pltpu.sync_copy(acc.at[sl], cnt_hbm.at[c, sl])


@functools.partial(
    pl.kernel,
    mesh=_mesh,
    out_type=jax.ShapeDtypeStruct((NC, NPAD, H), jnp.float32),
    scratch_types=[
        pltpu.VMEM((NBPT, EB), jnp.int32),
        pltpu.VMEM((EB,), jnp.int32),
        pltpu.VMEM((EB,), jnp.int32),
        pltpu.VMEM((EB, H), jnp.float32),
        pltpu.VMEM((EB, H), jnp.float32),
        pltpu.VMEM_SHARED((NPAD, H), jnp.float32),
        pltpu.SemaphoreType.DMA,
        pltpu.SemaphoreType.DMA,
        pltpu.SemaphoreType.DMA,
        pltpu.SemaphoreType.DMA,
    ],
)
def _scat_kernel(z_hbm, src_hbm, dst_hbm, zeros_hbm, out_hbm,
                 sidx_all, didx0, didx1, rows0, rows1, acc,
                 semg0, semg1, semd0, semd1):
    c = lax.axis_index("c")
    s = lax.axis_index("s")
    wid = _wid()
    base = s * RPT
    for off, ln in _CHUNKS:
        pltpu.sync_copy(zeros_hbm.at[pl.ds(0, ln)],
                        acc.at[pl.ds(base + off, ln)])
    pltpu.sync_copy(src_hbm.at[pl.ds(wid * NBPT, NBPT)], sidx_all)
    plsc.subcore_barrier()

    gbase = wid * NBPT

    def pair(i, carry):
        j0 = 2 * i
        c0 = pltpu.async_copy(z_hbm.at[sidx_all.at[j0]], rows0, semg0)
        c1 = pltpu.async_copy(z_hbm.at[sidx_all.at[j0 + 1]], rows1, semg1)
        pltpu.sync_copy(dst_hbm.at[gbase + j0], didx0)
        pltpu.sync_copy(dst_hbm.at[gbase + j0 + 1], didx1)
        c0.wait()
        pltpu.sync_copy(rows0, acc.at[didx0], add=True)
        c1.wait()
        pltpu.sync_copy(rows1, acc.at[didx1], add=True)
        return carry

    lax.fori_loop(0, NBPT // 2, pair, 0)
    plsc.subcore_barrier()
    for off, ln in _CHUNKS:
        sl = pl.ds(base + off, ln)
        pltpu.sync_copy(acc.at[sl], out_hbm.at[c, sl])


def _mm1_body(x_ref, w_ref, cnt_ref, z_ref, dinv_ref):
    cnt = cnt_ref[0, :N, 0:1] + cnt_ref[1, :N, 0:1]
    dinv = lax.rsqrt(cnt + 1.0)
    dinv_ref[...] = dinv
    xw = jnp.dot(x_ref[...], w_ref[...], preferred_element_type=jnp.float32)
    z_ref[pl.ds(0, N), :] = xw * dinv
    z_ref[pl.ds(N, NPAD - N), :] = jnp.zeros((NPAD - N, H), jnp.float32)


_mm1 = functools.partial(
    pl.pallas_call,
    out_shape=(jax.ShapeDtypeStruct((NPAD, H), jnp.float32),
               jax.ShapeDtypeStruct((N, 1), jnp.float32)),
)(_mm1_body)


def _mid_body(p_ref, z1_ref, dinv_ref, w2_ref, b1_ref, z2_ref):
    s = p_ref[0, :N, :] + p_ref[1, :N, :]
    dinv = dinv_ref[...]
    h = jnp.maximum((s + z1_ref[:N, :]) * dinv + b1_ref[...], 0.0)
    z2_ref[pl.ds(0, N), :] = jnp.dot(
        h, w2_ref[...], preferred_element_type=jnp.float32) * dinv
    z2_ref[pl.ds(N, NPAD - N), :] = jnp.zeros((NPAD - N, H), jnp.float32)


_mid = functools.partial(
    pl.pallas_call,
    out_shape=jax.ShapeDtypeStruct((NPAD, H), jnp.float32),
)(_mid_body)


def _fin_body(p_ref, z2_ref, dinv_ref, b2_ref, batch_ref, wl_ref, bl_ref,
              out_ref):
    s = p_ref[0, :N, :] + p_ref[1, :N, :]
    h = (s + z2_ref[:N, :]) * dinv_ref[...] + b2_ref[...]
    b = batch_ref[...]
    gids = lax.broadcasted_iota(jnp.int32, (G, N), 0)
    onehot = (b == gids).astype(jnp.float32)
    sums = jnp.dot(onehot, h, preferred_element_type=jnp.float32)
    cnts = jnp.sum(onehot, axis=1, keepdims=True)
    pooled = sums / jnp.maximum(cnts, 1.0)
    out_ref[...] = jnp.dot(
        pooled, wl_ref[...], preferred_element_type=jnp.float32) + bl_ref[...]


_fin = functools.partial(
    pl.pallas_call,
    out_shape=jax.ShapeDtypeStruct((G, 1), jnp.float32),
)(_fin_body)


def kernel(x, edge_index, batch, W1, b1, W2, b2, Wl, bl):
    pad = jnp.full((2, EPAD), DUMP, jnp.int32)
    ei = jnp.concatenate([edge_index, pad], axis=1)
    src = ei[0].reshape(NB2, EB)
    dst = ei[1].reshape(NB2, EB)
    ones128 = jnp.ones((EB, H), jnp.float32)
    zeros128 = jnp.zeros((EB, H), jnp.float32)

    cnt = _deg_kernel(dst, ones128, zeros128)
    z1, dinv = _mm1(x, W1, cnt)
    p1 = _scat_kernel(z1, src, dst, zeros128)
    z2 = _mid(p1, z1, dinv, W2, b1.reshape(1, H))
    p2 = _scat_kernel(z2, src, dst, zeros128)
    return _fin(p2, z2, dinv, b2.reshape(1, H), batch.reshape(1, N),
                Wl, bl.reshape(1, 1))


# speedup vs baseline: 1.0059x; 1.0059x over previous
import functools

import jax
import jax.numpy as jnp
from jax import lax
from jax.experimental import pallas as pl
from jax.experimental.pallas import tpu as pltpu
from jax.experimental.pallas import tpu_sc as plsc

N = 10000
E = 320000
D = 128
H = 128
G = 64

NC = 2
NS = 16
NW = NC * NS
EB = 128
NBPT = 80
NB2 = NBPT * NW
EPAD = NB2 * EB - E
NPAD = 10112
RPT = NPAD // NS
_CHUNKS = [(k * EB, EB) for k in range(RPT // EB)] + \
    ([(RPT - RPT % EB, RPT % EB)] if RPT % EB else [])
DUMP = NPAD - 1

_mesh = plsc.VectorSubcoreMesh(core_axis_name="c", subcore_axis_name="s")


def _wid():
    return lax.axis_index("s") * NC + lax.axis_index("c")


@functools.partial(
    pl.kernel,
    mesh=_mesh,
    out_type=jax.ShapeDtypeStruct((NC, NPAD, H), jnp.float32),
    scratch_types=[
        pltpu.VMEM((EB, H), jnp.float32),
        pltpu.VMEM((NBPT, EB), jnp.int32),
        pltpu.VMEM_SHARED((NPAD, H), jnp.float32),
    ],
)
def _deg_kernel(dst_hbm, ones_hbm, zeros_hbm, cnt_hbm, ones_v, didx_all,
                acc):
    c = lax.axis_index("c")
    s = lax.axis_index("s")
    wid = _wid()
    base = s * RPT
    for off, ln in _CHUNKS:
        pltpu.sync_copy(zeros_hbm.at[pl.ds(0, ln)],
                        acc.at[pl.ds(base + off, ln)])
    pltpu.sync_copy(ones_hbm, ones_v)
    pltpu.sync_copy(dst_hbm.at[pl.ds(wid * NBPT, NBPT)], didx_all)
    plsc.subcore_barrier()

    def body(j, carry):
        pltpu.sync_copy(ones_v, acc.at[didx_all.at[j]], add=True)
        return carry

    lax.fori_loop(0, NBPT, body, 0)
    plsc.subcore_barrier()
    for off, ln in _CHUNKS:
        sl = pl.ds(base + off, ln)
        pltpu.sync_copy(acc.at[sl], cnt_hbm.at[c, sl])


@functools.partial(
    pl.kernel,
    mesh=_mesh,
    out_type=jax.ShapeDtypeStruct((NC, NPAD, H), jnp.float32),
    scratch_types=[
        pltpu.VMEM((EB,), jnp.int32),
        pltpu.VMEM((EB,), jnp.int32),
        pltpu.VMEM((EB,), jnp.int32),
        pltpu.VMEM((EB,), jnp.int32),
        pltpu.VMEM((EB, H), jnp.float32),
        pltpu.VMEM((EB, H), jnp.float32),
        pltpu.VMEM_SHARED((NPAD, H), jnp.float32),
        pltpu.SemaphoreType.DMA,
        pltpu.SemaphoreType.DMA,
        pltpu.SemaphoreType.DMA,
        pltpu.SemaphoreType.DMA,
    ],
)
def _scat_kernel(z_hbm, src_hbm, dst_hbm, zeros_hbm, out_hbm,
                 sidx0, sidx1, didx0, didx1, rows0, rows1, acc,
                 semg0, semg1, semd0, semd1):
    c = lax.axis_index("c")
    s = lax.axis_index("s")
    wid = _wid()
    base = s * RPT
    for off, ln in _CHUNKS:
        pltpu.sync_copy(zeros_hbm.at[pl.ds(0, ln)],
                        acc.at[pl.ds(base + off, ln)])
    plsc.subcore_barrier()

    gbase = wid * NBPT

    def pair(i, carry):
        j0 = 2 * i
        pltpu.sync_copy(src_hbm.at[gbase + j0], sidx0)
        c0 = pltpu.async_copy(z_hbm.at[sidx0], rows0, semg0)
        pltpu.sync_copy(src_hbm.at[gbase + j0 + 1], sidx1)
        c1 = pltpu.async_copy(z_hbm.at[sidx1], rows1, semg1)
        pltpu.sync_copy(dst_hbm.at[gbase + j0], didx0)
        pltpu.sync_copy(dst_hbm.at[gbase + j0 + 1], didx1)
        c0.wait()
        pltpu.sync_copy(rows0, acc.at[didx0], add=True)
        c1.wait()
        pltpu.sync_copy(rows1, acc.at[didx1], add=True)
        return carry

    lax.fori_loop(0, NBPT // 2, pair, 0)
    plsc.subcore_barrier()
    for off, ln in _CHUNKS:
        sl = pl.ds(base + off, ln)
        pltpu.sync_copy(acc.at[sl], out_hbm.at[c, sl])


def _mm1_body(x_ref, w_ref, cnt_ref, z_ref, dinv_ref):
    cnt = cnt_ref[0, :N, 0:1] + cnt_ref[1, :N, 0:1]
    dinv = lax.rsqrt(cnt + 1.0)
    dinv_ref[...] = dinv
    xw = jnp.dot(x_ref[...], w_ref[...], preferred_element_type=jnp.float32)
    z_ref[pl.ds(0, N), :] = xw * dinv
    z_ref[pl.ds(N, NPAD - N), :] = jnp.zeros((NPAD - N, H), jnp.float32)


_mm1 = functools.partial(
    pl.pallas_call,
    out_shape=(jax.ShapeDtypeStruct((NPAD, H), jnp.float32),
               jax.ShapeDtypeStruct((N, 1), jnp.float32)),
)(_mm1_body)


def _mid_body(p_ref, z1_ref, dinv_ref, w2_ref, b1_ref, z2_ref):
    s = p_ref[0, :N, :] + p_ref[1, :N, :]
    dinv = dinv_ref[...]
    h = jnp.maximum((s + z1_ref[:N, :]) * dinv + b1_ref[...], 0.0)
    z2_ref[pl.ds(0, N), :] = jnp.dot(
        h, w2_ref[...], preferred_element_type=jnp.float32) * dinv
    z2_ref[pl.ds(N, NPAD - N), :] = jnp.zeros((NPAD - N, H), jnp.float32)


_mid = functools.partial(
    pl.pallas_call,
    out_shape=jax.ShapeDtypeStruct((NPAD, H), jnp.float32),
)(_mid_body)


def _fin_body(p_ref, z2_ref, dinv_ref, b2_ref, batch_ref, wl_ref, bl_ref,
              out_ref):
    s = p_ref[0, :N, :] + p_ref[1, :N, :]
    h = (s + z2_ref[:N, :]) * dinv_ref[...] + b2_ref[...]
    b = batch_ref[...]
    gids = lax.broadcasted_iota(jnp.int32, (G, N), 0)
    onehot = (b == gids).astype(jnp.float32)
    sums = jnp.dot(onehot, h, preferred_element_type=jnp.float32)
    cnts = jnp.sum(onehot, axis=1, keepdims=True)
    pooled = sums / jnp.maximum(cnts, 1.0)
    out_ref[...] = jnp.dot(
        pooled, wl_ref[...], preferred_element_type=jnp.float32) + bl_ref[...]


_fin = functools.partial(
    pl.pallas_call,
    out_shape=jax.ShapeDtypeStruct((G, 1), jnp.float32),
)(_fin_body)


def kernel(x, edge_index, batch, W1, b1, W2, b2, Wl, bl):
    pad = jnp.full((2, EPAD), DUMP, jnp.int32)
    ei = jnp.concatenate([edge_index, pad], axis=1)
    src = ei[0].reshape(NB2, EB)
    dst = ei[1].reshape(NB2, EB)
    ones128 = jnp.ones((EB, H), jnp.float32)
    zeros128 = jnp.zeros((EB, H), jnp.float32)

    cnt = _deg_kernel(dst, ones128, zeros128)
    z1, dinv = _mm1(x, W1, cnt)
    p1 = _scat_kernel(z1, src, dst, zeros128)
    z2 = _mid(p1, z1, dinv, W2, b1.reshape(1, H))
    p2 = _scat_kernel(z2, src, dst, zeros128)
    return _fin(p2, z2, dinv, b2.reshape(1, H), batch.reshape(1, N),
                Wl, bl.reshape(1, 1))


# speedup vs baseline: 2.7863x; 2.7700x over previous
import functools

import jax
import jax.numpy as jnp
from jax import lax
from jax.experimental import pallas as pl
from jax.experimental.pallas import tpu as pltpu
from jax.experimental.pallas import tpu_sc as plsc

N = 10000
E = 320000
D = 128
H = 128
G = 64

NC = 2
NS = 16
NW = NC * NS
EB = 128
NBPT = 80
NB2 = NBPT * NW
EPAD = NB2 * EB - E
NPAD = 10112
RPT = NPAD // NS
_CHUNKS = [(k * EB, EB) for k in range(RPT // EB)] + \
    ([(RPT - RPT % EB, RPT % EB)] if RPT % EB else [])
DUMP = NPAD - 1

_mesh = plsc.VectorSubcoreMesh(core_axis_name="c", subcore_axis_name="s")


def _wid():
    return lax.axis_index("s") * NC + lax.axis_index("c")


@functools.partial(
    pl.kernel,
    mesh=_mesh,
    out_type=jax.ShapeDtypeStruct((NC, NPAD, H), jnp.float32),
    scratch_types=[
        pltpu.VMEM((EB, H), jnp.float32),
        pltpu.VMEM((NBPT, EB), jnp.int32),
        pltpu.VMEM_SHARED((NPAD, H), jnp.float32),
    ],
)
def _deg_kernel(dst_hbm, ones_hbm, zeros_hbm, cnt_hbm, ones_v, didx_all,
                acc):
    c = lax.axis_index("c")
    s = lax.axis_index("s")
    wid = _wid()
    base = s * RPT
    for off, ln in _CHUNKS:
        pltpu.sync_copy(zeros_hbm.at[pl.ds(0, ln)],
                        acc.at[pl.ds(base + off, ln)])
    pltpu.sync_copy(ones_hbm, ones_v)
    pltpu.sync_copy(dst_hbm.at[pl.ds(wid * NBPT, NBPT)], didx_all)
    plsc.subcore_barrier()

    def body(j, carry):
        pltpu.sync_copy(ones_v, acc.at[didx_all.at[j]], add=True)
        return carry

    lax.fori_loop(0, NBPT, body, 0)
    plsc.subcore_barrier()
    for off, ln in _CHUNKS:
        sl = pl.ds(base + off, ln)
        pltpu.sync_copy(acc.at[sl], cnt_hbm.at[c, sl])


@functools.partial(
    pl.kernel,
    mesh=_mesh,
    out_type=jax.ShapeDtypeStruct((NC, NPAD, H), jnp.float32),
    scratch_types=[
        pltpu.VMEM((EB,), jnp.int32),
        pltpu.VMEM((EB,), jnp.int32),
        pltpu.VMEM((EB,), jnp.int32),
        pltpu.VMEM((EB,), jnp.int32),
        pltpu.VMEM((EB, H), jnp.float32),
        pltpu.VMEM((EB, H), jnp.float32),
        pltpu.VMEM_SHARED((NPAD, H), jnp.float32),
        pltpu.SemaphoreType.DMA,
        pltpu.SemaphoreType.DMA,
        pltpu.SemaphoreType.DMA,
        pltpu.SemaphoreType.DMA,
    ],
)
def _scat_kernel(z_hbm, src_hbm, dst_hbm, zeros_hbm, out_hbm,
                 sidx0, sidx1, didx0, didx1, rows0, rows1, acc,
                 semg0, semg1, semd0, semd1):
    c = lax.axis_index("c")
    s = lax.axis_index("s")
    wid = _wid()
    base = s * RPT
    for off, ln in _CHUNKS:
        pltpu.sync_copy(zeros_hbm.at[pl.ds(0, ln)],
                        acc.at[pl.ds(base + off, ln)])
    plsc.subcore_barrier()

    gbase = wid * NBPT

    def pair(i, carry):
        j0 = 2 * i
        pltpu.sync_copy(src_hbm.at[gbase + j0], sidx0)
        c0 = pltpu.async_copy(z_hbm.at[sidx0], rows0, semg0)
        pltpu.sync_copy(src_hbm.at[gbase + j0 + 1], sidx1)
        c1 = pltpu.async_copy(z_hbm.at[sidx1], rows1, semg1)
        pltpu.sync_copy(dst_hbm.at[gbase + j0], didx0)
        pltpu.sync_copy(dst_hbm.at[gbase + j0 + 1], didx1)
        c0.wait()
        pltpu.sync_copy(rows0, acc.at[didx0], add=True)
        c1.wait()
        pltpu.sync_copy(rows1, acc.at[didx1], add=True)
        return carry

    lax.fori_loop(0, NBPT // 2, pair, 0)
    plsc.subcore_barrier()
    for off, ln in _CHUNKS:
        sl = pl.ds(base + off, ln)
        pltpu.sync_copy(acc.at[sl], out_hbm.at[c, sl])


def _mm1_body(x_ref, w_ref, cnt_ref, z_ref, dinv_ref):
    cnt = cnt_ref[0, :N, 0:1] + cnt_ref[1, :N, 0:1]
    dinv = lax.rsqrt(cnt + 1.0)
    dinv_ref[...] = dinv
    xw = jnp.dot(x_ref[...], w_ref[...], preferred_element_type=jnp.float32)
    z_ref[pl.ds(0, N), :] = xw * dinv
    z_ref[pl.ds(N, NPAD - N), :] = jnp.zeros((NPAD - N, H), jnp.float32)


_mm1 = functools.partial(
    pl.pallas_call,
    out_shape=(jax.ShapeDtypeStruct((NPAD, H), jnp.float32),
               jax.ShapeDtypeStruct((N, 1), jnp.float32)),
)(_mm1_body)


def _mid_body(p_ref, z1_ref, dinv_ref, w2_ref, b1_ref, z2_ref):
    s = p_ref[0, :N, :] + p_ref[1, :N, :]
    dinv = dinv_ref[...]
    h = jnp.maximum((s + z1_ref[:N, :]) * dinv + b1_ref[...], 0.0)
    z2_ref[pl.ds(0, N), :] = jnp.dot(
        h, w2_ref[...], preferred_element_type=jnp.float32) * dinv
    z2_ref[pl.ds(N, NPAD - N), :] = jnp.zeros((NPAD - N, H), jnp.float32)


_mid = functools.partial(
    pl.pallas_call,
    out_shape=jax.ShapeDtypeStruct((NPAD, H), jnp.float32),
)(_mid_body)


def _fin_body(p_ref, z2_ref, dinv_ref, b2_ref, batch_ref, wl_ref, bl_ref,
              out_ref):
    s = p_ref[0, :N, :] + p_ref[1, :N, :]
    h = (s + z2_ref[:N, :]) * dinv_ref[...] + b2_ref[...]
    b = batch_ref[...]
    gids = lax.broadcasted_iota(jnp.int32, (G, N), 0)
    onehot = (b == gids).astype(jnp.float32)
    sums = jnp.dot(onehot, h, preferred_element_type=jnp.float32)
    cnts = jnp.sum(onehot, axis=1, keepdims=True)
    pooled = sums / jnp.maximum(cnts, 1.0)
    out_ref[...] = jnp.dot(
        pooled, wl_ref[...], preferred_element_type=jnp.float32) + bl_ref[...]


_fin = functools.partial(
    pl.pallas_call,
    out_shape=jax.ShapeDtypeStruct((G, 1), jnp.float32),
)(_fin_body)


def kernel(x, edge_index, batch, W1, b1, W2, b2, Wl, bl):
    lane = jnp.arange(EPAD, dtype=jnp.int32)
    src = jnp.concatenate([edge_index[0], lane % N]).reshape(NB2, EB)
    dst = jnp.concatenate([edge_index[1], N + lane % (NPAD - N)]
                          ).reshape(NB2, EB)
    ones128 = jnp.ones((EB, H), jnp.float32)
    zeros128 = jnp.zeros((EB, H), jnp.float32)

    cnt = _deg_kernel(dst, ones128, zeros128)
    z1, dinv = _mm1(x, W1, cnt)
    p1 = _scat_kernel(z1, src, dst, zeros128)
    z2 = _mid(p1, z1, dinv, W2, b1.reshape(1, H))
    p2 = _scat_kernel(z2, src, dst, zeros128)
    return _fin(p2, z2, dinv, b2.reshape(1, H), batch.reshape(1, N),
                Wl, bl.reshape(1, 1))


# speedup vs baseline: 2.8984x; 1.0402x over previous
import functools

import jax
import jax.numpy as jnp
from jax import lax
from jax.experimental import pallas as pl
from jax.experimental.pallas import tpu as pltpu
from jax.experimental.pallas import tpu_sc as plsc

N = 10000
E = 320000
D = 128
H = 128
G = 64

NC = 2
NS = 16
NW = NC * NS
EB = 128
NBPT = 80
NB2 = NBPT * NW
EPAD = NB2 * EB - E
NPAD = 10112
RPT = NPAD // NS
_CHUNKS = [(k * EB, EB) for k in range(RPT // EB)] + \
    ([(RPT - RPT % EB, RPT % EB)] if RPT % EB else [])
DUMP = NPAD - 1

_mesh = plsc.VectorSubcoreMesh(core_axis_name="c", subcore_axis_name="s")


def _wid():
    return lax.axis_index("s") * NC + lax.axis_index("c")


@functools.partial(
    pl.kernel,
    mesh=_mesh,
    out_type=jax.ShapeDtypeStruct((NC, NPAD, H), jnp.float32),
    scratch_types=[
        pltpu.VMEM((EB, H), jnp.float32),
        pltpu.VMEM((NBPT, EB), jnp.int32),
        pltpu.VMEM_SHARED((NPAD, H), jnp.float32),
    ],
)
def _deg_kernel(dst_hbm, ones_hbm, zeros_hbm, cnt_hbm, ones_v, didx_all,
                acc):
    c = lax.axis_index("c")
    s = lax.axis_index("s")
    wid = _wid()
    base = s * RPT
    for off, ln in _CHUNKS:
        pltpu.sync_copy(zeros_hbm.at[pl.ds(0, ln)],
                        acc.at[pl.ds(base + off, ln)])
    pltpu.sync_copy(ones_hbm, ones_v)
    pltpu.sync_copy(dst_hbm.at[pl.ds(wid * NBPT, NBPT)], didx_all)
    plsc.subcore_barrier()

    def body(j, carry):
        pltpu.sync_copy(ones_v, acc.at[didx_all.at[j]], add=True)
        return carry

    lax.fori_loop(0, NBPT, body, 0)
    plsc.subcore_barrier()
    for off, ln in _CHUNKS:
        sl = pl.ds(base + off, ln)
        pltpu.sync_copy(acc.at[sl], cnt_hbm.at[c, sl])


@functools.partial(
    pl.kernel,
    mesh=_mesh,
    out_type=jax.ShapeDtypeStruct((NC, NPAD, H), jnp.float32),
    scratch_types=[
        pltpu.VMEM((NBPT, EB), jnp.int32),
        pltpu.VMEM((EB,), jnp.int32),
        pltpu.VMEM((EB,), jnp.int32),
        pltpu.VMEM((EB, H), jnp.float32),
        pltpu.VMEM((EB, H), jnp.float32),
        pltpu.VMEM_SHARED((NPAD, H), jnp.float32),
        pltpu.SemaphoreType.DMA,
        pltpu.SemaphoreType.DMA,
        pltpu.SemaphoreType.DMA,
        pltpu.SemaphoreType.DMA,
    ],
)
def _scat_kernel(z_hbm, src_hbm, dst_hbm, zeros_hbm, out_hbm,
                 sidx_all, didx0, didx1, rows0, rows1, acc,
                 semg0, semg1, semd0, semd1):
    c = lax.axis_index("c")
    s = lax.axis_index("s")
    wid = _wid()
    base = s * RPT
    for off, ln in _CHUNKS:
        pltpu.sync_copy(zeros_hbm.at[pl.ds(0, ln)],
                        acc.at[pl.ds(base + off, ln)])
    pltpu.sync_copy(src_hbm.at[pl.ds(wid * NBPT, NBPT)], sidx_all)
    plsc.subcore_barrier()

    gbase = wid * NBPT

    def pair(i, carry):
        j0 = 2 * i
        c0 = pltpu.async_copy(z_hbm.at[sidx_all.at[j0]], rows0, semg0)
        c1 = pltpu.async_copy(z_hbm.at[sidx_all.at[j0 + 1]], rows1, semg1)
        pltpu.sync_copy(dst_hbm.at[gbase + j0], didx0)
        pltpu.sync_copy(dst_hbm.at[gbase + j0 + 1], didx1)
        c0.wait()
        pltpu.sync_copy(rows0, acc.at[didx0], add=True)
        c1.wait()
        pltpu.sync_copy(rows1, acc.at[didx1], add=True)
        return carry

    lax.fori_loop(0, NBPT // 2, pair, 0)
    plsc.subcore_barrier()
    for off, ln in _CHUNKS:
        sl = pl.ds(base + off, ln)
        pltpu.sync_copy(acc.at[sl], out_hbm.at[c, sl])


def _mm1_body(x_ref, w_ref, cnt_ref, z_ref, dinv_ref):
    cnt = cnt_ref[0, :N, 0:1] + cnt_ref[1, :N, 0:1]
    dinv = lax.rsqrt(cnt + 1.0)
    dinv_ref[...] = dinv
    xw = jnp.dot(x_ref[...], w_ref[...], preferred_element_type=jnp.float32)
    z_ref[pl.ds(0, N), :] = xw * dinv
    z_ref[pl.ds(N, NPAD - N), :] = jnp.zeros((NPAD - N, H), jnp.float32)


_mm1 = functools.partial(
    pl.pallas_call,
    out_shape=(jax.ShapeDtypeStruct((NPAD, H), jnp.float32),
               jax.ShapeDtypeStruct((N, 1), jnp.float32)),
)(_mm1_body)


def _mid_body(p_ref, z1_ref, dinv_ref, w2_ref, b1_ref, z2_ref):
    s = p_ref[0, :N, :] + p_ref[1, :N, :]
    dinv = dinv_ref[...]
    h = jnp.maximum((s + z1_ref[:N, :]) * dinv + b1_ref[...], 0.0)
    z2_ref[pl.ds(0, N), :] = jnp.dot(
        h, w2_ref[...], preferred_element_type=jnp.float32) * dinv
    z2_ref[pl.ds(N, NPAD - N), :] = jnp.zeros((NPAD - N, H), jnp.float32)


_mid = functools.partial(
    pl.pallas_call,
    out_shape=jax.ShapeDtypeStruct((NPAD, H), jnp.float32),
)(_mid_body)


def _fin_body(p_ref, z2_ref, dinv_ref, b2_ref, batch_ref, wl_ref, bl_ref,
              out_ref):
    s = p_ref[0, :N, :] + p_ref[1, :N, :]
    h = (s + z2_ref[:N, :]) * dinv_ref[...] + b2_ref[...]
    b = batch_ref[...]
    gids = lax.broadcasted_iota(jnp.int32, (G, N), 0)
    onehot = (b == gids).astype(jnp.float32)
    sums = jnp.dot(onehot, h, preferred_element_type=jnp.float32)
    cnts = jnp.sum(onehot, axis=1, keepdims=True)
    pooled = sums / jnp.maximum(cnts, 1.0)
    out_ref[...] = jnp.dot(
        pooled, wl_ref[...], preferred_element_type=jnp.float32) + bl_ref[...]


_fin = functools.partial(
    pl.pallas_call,
    out_shape=jax.ShapeDtypeStruct((G, 1), jnp.float32),
)(_fin_body)


def kernel(x, edge_index, batch, W1, b1, W2, b2, Wl, bl):
    lane = jnp.arange(EPAD, dtype=jnp.int32)
    src = jnp.concatenate([edge_index[0], lane % N]).reshape(NB2, EB)
    dst = jnp.concatenate([edge_index[1], N + lane % (NPAD - N)]
                          ).reshape(NB2, EB)
    ones128 = jnp.ones((EB, H), jnp.float32)
    zeros128 = jnp.zeros((EB, H), jnp.float32)

    cnt = _deg_kernel(dst, ones128, zeros128)
    z1, dinv = _mm1(x, W1, cnt)
    p1 = _scat_kernel(z1, src, dst, zeros128)
    z2 = _mid(p1, z1, dinv, W2, b1.reshape(1, H))
    p2 = _scat_kernel(z2, src, dst, zeros128)
    return _fin(p2, z2, dinv, b2.reshape(1, H), batch.reshape(1, N),
                Wl, bl.reshape(1, 1))
